# baseline (speedup 1.0000x reference)
import jax
import jax.numpy as jnp
from jax.experimental import pallas as pl
from jax.experimental.pallas import tpu as pltpu

B = 128
IN_FEAT = 256
OUT_FEAT = 128
N = 38
EMBED_DIM = 128
ALPHA = 0.2
TOP_K = 10

BB = 32


def _gat_kernel(x_ref, w_ref, a_ref, emb_ref, out_ref, maskt_ref):
    @pl.when(pl.program_id(0) == 0)
    def _():
        emb = emb_ref[...]
        gram = jax.lax.dot_general(
            emb, emb, (((1,), (1,)), ((), ())),
            preferred_element_type=jnp.float32)
        nrm = jnp.sqrt(jnp.sum(emb * emb, axis=1, keepdims=True))
        adj = gram / (nrm * nrm.T)
        a1_ = adj[:, None, :]
        a2_ = adj[None, :, :]
        mdx = jax.lax.broadcasted_iota(jnp.int32, (N, N, N), 0)
        kdx = jax.lax.broadcasted_iota(jnp.int32, (N, N, N), 1)
        gt = (a1_ > a2_) | ((a1_ == a2_) & (mdx < kdx))
        rank = jnp.sum(gt.astype(jnp.float32), axis=0)
        sel = (rank == jnp.float32(TOP_K - 2)).astype(jnp.float32)
        thresh_t = jnp.sum(adj * sel, axis=0, keepdims=True)
        maskt = (adj > thresh_t) | (adj == jnp.float32(1.0))
        maskt_ref[...] = maskt.astype(jnp.float32)

    w = w_ref[...]
    a = a_ref[...]
    a1 = a[:OUT_FEAT, :]
    a2 = a[OUT_FEAT:, :]
    maskt = maskt_ref[...] > jnp.float32(0.5)

    hts = [
        jax.lax.dot_general(
            w, x_ref[b], (((0,), (0,)), ((), ())),
            preferred_element_type=jnp.float32)
        for b in range(BB)
    ]
    f1s = [
        jax.lax.dot_general(
            a1, ht, (((0,), (0,)), ((), ())),
            preferred_element_type=jnp.float32)
        for ht in hts
    ]
    f2s = [
        jax.lax.dot_general(
            ht, a2, (((0,), (0,)), ((), ())),
            preferred_element_type=jnp.float32)
        for ht in hts
    ]
    atts = []
    for b in range(BB):
        et = f2s[b] + f1s[b]
        et = jnp.where(et >= 0, et, jnp.float32(ALPHA) * et)
        att = jnp.where(maskt, et, jnp.float32(-1e12))
        att = att - jnp.max(att, axis=1, keepdims=True)
        att = jnp.exp(att)
        atts.append(att / jnp.sum(att, axis=1, keepdims=True))
    for b in range(BB):
        hp = jax.lax.dot_general(
            hts[b], atts[b], (((1,), (0,)), ((), ())),
            preferred_element_type=jnp.float32)
        out_ref[b] = jnp.where(hp > 0, hp, jnp.exp(hp) - 1.0)


@jax.jit
def kernel(x, W, a, emb):
    grid = (B // BB,)
    return pl.pallas_call(
        _gat_kernel,
        grid=grid,
        in_specs=[
            pl.BlockSpec((BB, IN_FEAT, N), lambda b: (b, 0, 0)),
            pl.BlockSpec((IN_FEAT, OUT_FEAT), lambda b: (0, 0)),
            pl.BlockSpec((2 * OUT_FEAT, 1), lambda b: (0, 0)),
            pl.BlockSpec((N, EMBED_DIM), lambda b: (0, 0)),
        ],
        out_specs=pl.BlockSpec((BB, OUT_FEAT, N), lambda b: (b, 0, 0)),
        out_shape=jax.ShapeDtypeStruct((B, OUT_FEAT, N), jnp.float32),
        scratch_shapes=[pltpu.VMEM((N, N), jnp.float32)],
    )(x, W, a, emb)
